# baseline (speedup 1.0000x reference)
import dataclasses

import jax
import jax.numpy as jnp
from jax import lax
from jax.experimental import pallas as pl
from jax.experimental.pallas import tpu as pltpu
from jax.experimental.pallas import tpu_sc as plsc

EMBED_DIM = 32
NBUF = 8
NUM_WORKERS = 32
TBLOCK = 800

_CP = pltpu.CompilerParams(use_tc_tiling_on_sc=False)
_CP_NL = _CP
if "needs_layout_passes" in pltpu.CompilerParams.__dataclass_fields__:
    _CP_NL = dataclasses.replace(_CP, needs_layout_passes=False)


def _transpose_table(table_t, vocab):
    nblk = vocab // TBLOCK
    per_worker = (nblk + NUM_WORKERS - 1) // NUM_WORKERS
    mesh = plsc.VectorSubcoreMesh(core_axis_name="core", subcore_axis_name="subcore")

    scratch = [
        pltpu.VMEM((EMBED_DIM, TBLOCK), jnp.float32),
        pltpu.VMEM((TBLOCK, EMBED_DIM), jnp.float32),
        pltpu.SemaphoreType.DMA,
        pltpu.SemaphoreType.DMA,
    ]

    @pl.kernel(
        out_type=jax.ShapeDtypeStruct((vocab, EMBED_DIM), jnp.float32),
        mesh=mesh,
        scratch_types=scratch,
        compiler_params=_CP_NL,
    )
    def transpose_kernel(tt_hbm, out_hbm, vin, vout, si, so):
        wid = lax.axis_index("subcore") * 2 + lax.axis_index("core")
        iota16 = lax.iota(jnp.int32, 16)
        cols = [jnp.full((16,), c, jnp.int32) for c in range(EMBED_DIM)]

        @pl.loop(0, per_worker)
        def _(kk):
            g = jnp.minimum(kk * NUM_WORKERS + wid, nblk - 1)
            pltpu.async_copy(tt_hbm.at[:, pl.ds(g * TBLOCK, TBLOCK)], vin, si).wait()

            @pl.loop(0, TBLOCK // 16)
            def _(j):
                rows = j * 16 + iota16
                for c in range(EMBED_DIM):
                    vals = vin[c, pl.ds(j * 16, 16)]
                    plsc.store_scatter(vout, [rows, cols[c]], vals)

            pltpu.async_copy(vout, out_hbm.at[pl.ds(g * TBLOCK, TBLOCK), :], so).wait()

    return transpose_kernel(table_t)


def kernel(x, table):
    batch, seq = x.shape
    vocab = table.shape[0]
    rows_per_worker = batch // NUM_WORKERS
    rounds = rows_per_worker // NBUF

    table_lin = _transpose_table(table.T, vocab)

    mesh = plsc.VectorSubcoreMesh(core_axis_name="core", subcore_axis_name="subcore")

    scratch = (
        [pltpu.VMEM((rows_per_worker, seq), jnp.int32)]
        + [pltpu.VMEM((seq, EMBED_DIM), jnp.float32) for _ in range(NBUF)]
        + [pltpu.SemaphoreType.DMA for _ in range(2 * NBUF + 1)]
    )

    @pl.kernel(
        out_type=jax.ShapeDtypeStruct((batch, seq, EMBED_DIM), table.dtype),
        mesh=mesh,
        scratch_types=scratch,
        compiler_params=_CP,
    )
    def gather_kernel(table_hbm, idx_hbm, out_hbm, idx_v, *rest):
        bufs = rest[:NBUF]
        gsem = rest[NBUF:2 * NBUF]
        wsem = rest[2 * NBUF:3 * NBUF]
        isem = rest[3 * NBUF]

        wid = lax.axis_index("subcore") * 2 + lax.axis_index("core")
        b0 = wid * rows_per_worker

        pltpu.async_copy(idx_hbm.at[pl.ds(b0, rows_per_worker), :], idx_v, isem).wait()

        def start_gather(j, b):
            pltpu.async_copy(table_hbm.at[idx_v.at[j]], bufs[b], gsem[b])

        def wait_gather(j, b):
            pltpu.make_async_copy(
                table_hbm.at[idx_v.at[j]], bufs[b], gsem[b]
            ).wait()

        def start_write(j, b):
            pltpu.async_copy(bufs[b], out_hbm.at[b0 + j], wsem[b])

        def wait_write(j, b):
            pltpu.make_async_copy(bufs[b], out_hbm.at[b0 + j], wsem[b]).wait()

        for b in range(NBUF):
            start_gather(b, b)

        @pl.loop(0, rounds - 1)
        def _(g):
            base = g * NBUF
            for b in range(NBUF):
                wait_gather(base + b, b)
                start_write(base + b, b)
            for b in range(NBUF):
                wait_write(base + b, b)
                start_gather(base + NBUF + b, b)

        base = (rounds - 1) * NBUF
        for b in range(NBUF):
            wait_gather(base + b, b)
            start_write(base + b, b)
        for b in range(NBUF):
            wait_write(base + b, b)

    return gather_kernel(table_lin, x)


# speedup vs baseline: 3.6519x; 3.6519x over previous
import jax
import jax.numpy as jnp
from jax import lax
from jax.experimental import pallas as pl
from jax.experimental.pallas import tpu as pltpu
from jax.experimental.pallas import tpu_sc as plsc

EMBED_DIM = 32
NBUF = 8
NUM_WORKERS = 32


def kernel(x, table):
    batch, seq = x.shape
    rows_per_worker = batch // NUM_WORKERS
    rounds = rows_per_worker // NBUF

    mesh = plsc.VectorSubcoreMesh(core_axis_name="core", subcore_axis_name="subcore")

    scratch = (
        [pltpu.VMEM((rows_per_worker, seq), jnp.int32)]
        + [pltpu.VMEM((seq, EMBED_DIM), jnp.float32) for _ in range(NBUF)]
        + [pltpu.SemaphoreType.DMA for _ in range(2 * NBUF + 1)]
    )

    @pl.kernel(
        out_type=jax.ShapeDtypeStruct((batch, seq, EMBED_DIM), table.dtype),
        mesh=mesh,
        scratch_types=scratch,
        compiler_params=pltpu.CompilerParams(use_tc_tiling_on_sc=False),
    )
    def gather_kernel(table_hbm, idx_hbm, out_hbm, idx_v, *rest):
        bufs = rest[:NBUF]
        gsem = rest[NBUF:2 * NBUF]
        wsem = rest[2 * NBUF:3 * NBUF]
        isem = rest[3 * NBUF]

        wid = lax.axis_index("subcore") * 2 + lax.axis_index("core")
        b0 = wid * rows_per_worker

        pltpu.async_copy(idx_hbm.at[pl.ds(b0, rows_per_worker), :], idx_v, isem).wait()

        def start_gather(j, b):
            pltpu.async_copy(table_hbm.at[idx_v.at[j]], bufs[b], gsem[b])

        def wait_gather(j, b):
            pltpu.make_async_copy(
                table_hbm.at[idx_v.at[j]], bufs[b], gsem[b]
            ).wait()

        def start_write(j, b):
            pltpu.async_copy(bufs[b], out_hbm.at[b0 + j], wsem[b])

        def wait_write(j, b):
            pltpu.make_async_copy(bufs[b], out_hbm.at[b0 + j], wsem[b]).wait()

        for b in range(NBUF):
            start_gather(b, b)

        @pl.loop(0, rounds - 1)
        def _(g):
            base = g * NBUF
            for b in range(NBUF):
                wait_gather(base + b, b)
                start_write(base + b, b)
            for b in range(NBUF):
                wait_write(base + b, b)
                start_gather(base + NBUF + b, b)

        base = (rounds - 1) * NBUF
        for b in range(NBUF):
            wait_gather(base + b, b)
            start_write(base + b, b)
        for b in range(NBUF):
            wait_write(base + b, b)

    return gather_kernel(table, x)
